# Initial kernel scaffold; baseline (speedup 1.0000x reference)
import jax
import jax.numpy as jnp
from jax.experimental import pallas as pl


def kernel(f0, phone_label, phone_duration, midi_label, W_f0, b_f0, phone_table, midi_table):
    raise NotImplementedError("write your pallas kernel here")


# speedup vs baseline: 1.5895x; 1.5895x over previous
import functools

import jax
import jax.numpy as jnp
from jax import lax
from jax.experimental import pallas as pl
from jax.experimental.pallas import tpu as pltpu
from jax.experimental.pallas import tpu_sc as plsc

B = 4096
S = 200
D_F0 = 64
NUM_PHONES = 100
NUM_MIDI = 128
D_OUT = 192
CHUNK_OFFS = tuple(range(0, S - 16, 16)) + (S - 16,)
MIDI_BASE = D_F0 * NUM_PHONES

NUM_WORKERS = 32
BPW = B // NUM_WORKERS


def _sc_encoder(f0_hbm, ph_hbm, md_hbm, tbl_hbm, wb_hbm, out_hbm,
                tbl_v, wb_v, f0_v, ph_v, md_v, out_v):
    wid = lax.axis_index("s") * 2 + lax.axis_index("c")
    base = wid * BPW

    pltpu.sync_copy(tbl_hbm, tbl_v)
    pltpu.sync_copy(wb_hbm, wb_v)

    def batch_body(i, carry):
        b = base + i
        pltpu.sync_copy(f0_hbm.at[b], f0_v)
        pltpu.sync_copy(ph_hbm.at[b], ph_v)
        pltpu.sync_copy(md_hbm.at[b], md_v)

        def d_body(d, c2):
            dv = jnp.full((16,), d, dtype=jnp.int32)
            wv = plsc.load_gather(wb_v, [dv])
            bv = plsc.load_gather(wb_v, [dv + D_F0])
            ph_off = d * NUM_PHONES
            md_off = MIDI_BASE + d * NUM_MIDI
            for off in CHUNK_OFFS:
                sl = pl.ds(off, 16)
                out_v[d, sl] = f0_v[sl] * wv + bv
                out_v[D_F0 + d, sl] = plsc.load_gather(tbl_v, [ph_v[sl] + ph_off])
                out_v[2 * D_F0 + d, sl] = plsc.load_gather(tbl_v, [md_v[sl] + md_off])
            return c2

        lax.fori_loop(0, D_F0, d_body, 0)
        pltpu.sync_copy(out_v, out_hbm.at[b])
        return carry

    lax.fori_loop(0, BPW, batch_body, 0)


def kernel(f0, phone_label, phone_duration, midi_label, W_f0, b_f0,
           phone_table, midi_table):
    del phone_duration
    tbl = jnp.concatenate([
        phone_table.T.reshape(-1),
        midi_table.T.reshape(-1),
    ]).astype(jnp.float32)
    wb = jnp.concatenate([W_f0[:, 0], b_f0]).astype(jnp.float32)
    ph = phone_label.astype(jnp.int32)
    md = midi_label.astype(jnp.int32)
    f0 = f0.astype(jnp.float32)

    mesh = plsc.VectorSubcoreMesh(core_axis_name="c", subcore_axis_name="s")
    run = functools.partial(
        pl.kernel,
        mesh=mesh,
        compiler_params=pltpu.CompilerParams(needs_layout_passes=False),
        out_type=jax.ShapeDtypeStruct((B, D_OUT, S), jnp.float32),
        scratch_types=[
            pltpu.VMEM((D_F0 * NUM_PHONES + D_F0 * NUM_MIDI,), jnp.float32),
            pltpu.VMEM((2 * D_F0,), jnp.float32),
            pltpu.VMEM((S,), jnp.float32),
            pltpu.VMEM((S,), jnp.int32),
            pltpu.VMEM((S,), jnp.int32),
            pltpu.VMEM((D_OUT, S), jnp.float32),
        ],
    )(_sc_encoder)
    return run(f0, ph, md, tbl, wb)


# speedup vs baseline: 1.9160x; 1.2054x over previous
import functools

import jax
import jax.numpy as jnp
from jax import lax
from jax.experimental import pallas as pl
from jax.experimental.pallas import tpu as pltpu
from jax.experimental.pallas import tpu_sc as plsc

B = 4096
S = 200
D_F0 = 64
NUM_PHONES = 100
NUM_MIDI = 128
D_OUT = 192
OUT_ROW = D_OUT * S
CHUNK_OFFS = tuple(range(0, S - 16, 16)) + (S - 16,)
NCHUNK = len(CHUNK_OFFS)
MIDI_BASE = D_F0 * NUM_PHONES

NUM_WORKERS = 32
BPW = B // NUM_WORKERS
G = 32
NGROUP = BPW // G


def _sc_encoder(f0_hbm, ph_hbm, md_hbm, tbl_hbm, wb_hbm, out_hbm,
                tbl_v, wb_v, wbc_v, bbc_v, f0g, phg, mdg, ob0, ob1,
                sem0, sem1):
    wid = lax.axis_index("s") * 2 + lax.axis_index("c")
    base = wid * BPW

    pltpu.sync_copy(tbl_hbm, tbl_v)
    pltpu.sync_copy(wb_hbm, wb_v)

    def bc_body(d, c):
        dv = jnp.full((16,), d, dtype=jnp.int32)
        wbc_v[pl.ds(d * 16, 16)] = plsc.load_gather(wb_v, [dv])
        bbc_v[pl.ds(d * 16, 16)] = plsc.load_gather(wb_v, [dv + D_F0])
        return c

    lax.fori_loop(0, D_F0, bc_body, 0)

    obs = (ob0, ob1)
    sems = (sem0, sem1)

    for g in range(NGROUP):
        gb0 = base + g * G
        pltpu.sync_copy(f0_hbm.at[pl.ds(gb0 * S, G * S)], f0g)
        pltpu.sync_copy(ph_hbm.at[pl.ds(gb0 * S, G * S)], phg)
        pltpu.sync_copy(md_hbm.at[pl.ds(gb0 * S, G * S)], mdg)

        def pair_body(i2, c, g=g):
            for j in (0, 1):
                li = i2 * 2 + j
                gl = g * G + li
                b = base + gl
                ob = obs[j]
                sem = sems[j]

                @pl.when(gl >= 2)
                def _():
                    pltpu.make_async_copy(
                        ob, out_hbm.at[pl.ds((b - 2) * OUT_ROW, OUT_ROW)], sem
                    ).wait()

                fbase = li * S

                def pass_a(d, c2, fbase=fbase, ob=ob):
                    wv = wbc_v[pl.ds(d * 16, 16)]
                    bv = bbc_v[pl.ds(d * 16, 16)]
                    obase = d * S
                    for off in CHUNK_OFFS:
                        ob[pl.ds(obase + off, 16)] = (
                            f0g[pl.ds(fbase + off, 16)] * wv + bv
                        )
                    return c2

                lax.fori_loop(0, D_F0, pass_a, 0)

                def pass_b(ci, c2, fbase=fbase, ob=ob):
                    off = jnp.minimum(ci * 16, S - 16)
                    lv = phg[pl.ds(fbase + off, 16)]
                    mv = mdg[pl.ds(fbase + off, 16)]
                    for d in range(D_F0):
                        ob[pl.ds((D_F0 + d) * S + off, 16)] = (
                            plsc.load_gather(tbl_v, [lv + d * NUM_PHONES])
                        )
                        ob[pl.ds((2 * D_F0 + d) * S + off, 16)] = (
                            plsc.load_gather(tbl_v, [mv + (MIDI_BASE + d * NUM_MIDI)])
                        )
                    return c2

                lax.fori_loop(0, NCHUNK, pass_b, 0)

                pltpu.async_copy(
                    ob, out_hbm.at[pl.ds(b * OUT_ROW, OUT_ROW)], sem
                )
            return c

        lax.fori_loop(0, G // 2, pair_body, 0)

    for j in (0, 1):
        bl = base + BPW - 2 + j
        pltpu.make_async_copy(
            obs[j], out_hbm.at[pl.ds(bl * OUT_ROW, OUT_ROW)], sems[j]
        ).wait()


def kernel(f0, phone_label, phone_duration, midi_label, W_f0, b_f0,
           phone_table, midi_table):
    del phone_duration
    tbl = jnp.concatenate([
        phone_table.T.reshape(-1),
        midi_table.T.reshape(-1),
    ]).astype(jnp.float32)
    wb = jnp.concatenate([W_f0[:, 0], b_f0]).astype(jnp.float32)
    ph = phone_label.astype(jnp.int32).reshape(-1)
    md = midi_label.astype(jnp.int32).reshape(-1)
    f0 = f0.astype(jnp.float32).reshape(-1)

    mesh = plsc.VectorSubcoreMesh(core_axis_name="c", subcore_axis_name="s")
    run = functools.partial(
        pl.kernel,
        mesh=mesh,
        compiler_params=pltpu.CompilerParams(needs_layout_passes=False),
        out_type=jax.ShapeDtypeStruct((B * OUT_ROW,), jnp.float32),
        scratch_types=[
            pltpu.VMEM((D_F0 * NUM_PHONES + D_F0 * NUM_MIDI,), jnp.float32),
            pltpu.VMEM((2 * D_F0,), jnp.float32),
            pltpu.VMEM((16 * D_F0,), jnp.float32),
            pltpu.VMEM((16 * D_F0,), jnp.float32),
            pltpu.VMEM((G * S,), jnp.float32),
            pltpu.VMEM((G * S,), jnp.int32),
            pltpu.VMEM((G * S,), jnp.int32),
            pltpu.VMEM((OUT_ROW,), jnp.float32),
            pltpu.VMEM((OUT_ROW,), jnp.float32),
            pltpu.SemaphoreType.DMA,
            pltpu.SemaphoreType.DMA,
        ],
    )(_sc_encoder)
    out = run(f0, ph, md, tbl, wb)
    return out.reshape(B, D_OUT, S)


# speedup vs baseline: 5.5455x; 2.8943x over previous
import functools

import jax
import jax.numpy as jnp
from jax import lax
from jax.experimental import pallas as pl
from jax.experimental.pallas import tpu as pltpu
from jax.experimental.pallas import tpu_sc as plsc

B = 4096
S = 200
D_F0 = 64
NUM_PHONES = 100
NUM_MIDI = 128
D_OUT = 192
CHUNK_OFFS = tuple(range(0, S - 16, 16)) + (S - 16,)
NCHUNK = len(CHUNK_OFFS)
MIDI_BASE = D_F0 * NUM_PHONES

NUM_WORKERS = 32
BPW = B // NUM_WORKERS
G = 16
NGROUP = BPW // G
DGRP = 8


def _sc_encoder(f0_hbm, ph_hbm, md_hbm, tbl_hbm, wb_hbm, out_hbm,
                tbl_v, wb_v, f0g, phg, mdg, ob0, ob1, sem0, sem1):
    wid = lax.axis_index("s") * 2 + lax.axis_index("c")
    base = wid * BPW

    pltpu.sync_copy(tbl_hbm, tbl_v)
    pltpu.sync_copy(wb_hbm, wb_v)

    obs = (ob0, ob1)
    sems = (sem0, sem1)

    for g in range(NGROUP):
        gb0 = base + g * G
        pltpu.sync_copy(f0_hbm.at[pl.ds(gb0, G)], f0g)
        pltpu.sync_copy(ph_hbm.at[pl.ds(gb0, G)], phg)
        pltpu.sync_copy(md_hbm.at[pl.ds(gb0, G)], mdg)

        def pair_body(i2, c, g=g):
            for j in (0, 1):
                li = i2 * 2 + j
                gl = g * G + li
                b = base + gl
                ob = obs[j]
                sem = sems[j]

                @pl.when(gl >= 2)
                def _():
                    pltpu.make_async_copy(ob, out_hbm.at[b - 2], sem).wait()

                f0cs = tuple(f0g[li, pl.ds(off, 16)] for off in CHUNK_OFFS)

                def pass_a(d, f0cs, ob=ob):
                    dv = jnp.full((16,), d, dtype=jnp.int32)
                    wv = plsc.load_gather(wb_v, [dv])
                    bv = plsc.load_gather(wb_v, [dv + D_F0])
                    for off, fc in zip(CHUNK_OFFS, f0cs):
                        ob[d, pl.ds(off, 16)] = fc * wv + bv
                    return f0cs

                lax.fori_loop(0, D_F0, pass_a, f0cs)

                def pass_b(ci, c2, li=li, ob=ob):
                    off = jnp.minimum(ci * 16, S - 16)
                    sl = pl.ds(off, 16)
                    lv = phg[li, sl]
                    mv = mdg[li, sl]
                    for d0 in range(0, D_F0, DGRP):
                        pv = [plsc.load_gather(tbl_v, [lv + d * NUM_PHONES])
                              for d in range(d0, d0 + DGRP)]
                        mvv = [plsc.load_gather(
                                   tbl_v, [mv + (MIDI_BASE + d * NUM_MIDI)])
                               for d in range(d0, d0 + DGRP)]
                        for k, d in enumerate(range(d0, d0 + DGRP)):
                            ob[D_F0 + d, sl] = pv[k]
                            ob[2 * D_F0 + d, sl] = mvv[k]
                    return c2

                lax.fori_loop(0, NCHUNK, pass_b, 0)

                pltpu.async_copy(ob, out_hbm.at[b], sem)
            return c

        lax.fori_loop(0, G // 2, pair_body, 0)

    for j in (0, 1):
        bl = base + BPW - 2 + j
        pltpu.make_async_copy(obs[j], out_hbm.at[bl], sems[j]).wait()


def kernel(f0, phone_label, phone_duration, midi_label, W_f0, b_f0,
           phone_table, midi_table):
    del phone_duration
    tbl = jnp.concatenate([
        phone_table.T.reshape(-1),
        midi_table.T.reshape(-1),
    ]).astype(jnp.float32)
    wb = jnp.concatenate([W_f0[:, 0], b_f0]).astype(jnp.float32)
    ph = phone_label.astype(jnp.int32)
    md = midi_label.astype(jnp.int32)
    f0 = f0.astype(jnp.float32)

    mesh = plsc.VectorSubcoreMesh(core_axis_name="c", subcore_axis_name="s")
    run = functools.partial(
        pl.kernel,
        mesh=mesh,
        compiler_params=pltpu.CompilerParams(needs_layout_passes=False),
        out_type=jax.ShapeDtypeStruct((B, D_OUT, S), jnp.float32),
        scratch_types=[
            pltpu.VMEM((D_F0 * NUM_PHONES + D_F0 * NUM_MIDI,), jnp.float32),
            pltpu.VMEM((2 * D_F0,), jnp.float32),
            pltpu.VMEM((G, S), jnp.float32),
            pltpu.VMEM((G, S), jnp.int32),
            pltpu.VMEM((G, S), jnp.int32),
            pltpu.VMEM((D_OUT, S), jnp.float32),
            pltpu.VMEM((D_OUT, S), jnp.float32),
            pltpu.SemaphoreType.DMA,
            pltpu.SemaphoreType.DMA,
        ],
    )(_sc_encoder)
    return run(f0, ph, md, tbl, wb)


# speedup vs baseline: 17.5467x; 3.1641x over previous
import functools

import jax
import jax.numpy as jnp
from jax import lax
from jax.experimental import pallas as pl
from jax.experimental.pallas import tpu as pltpu
from jax.experimental.pallas import tpu_sc as plsc

B = 4096
S = 200
D_F0 = 64
NUM_PHONES = 100
NUM_MIDI = 128
D_OUT = 192
MIDI_BASE = D_F0 * NUM_PHONES

NUM_WORKERS = 32
BC = B // NUM_WORKERS
NBCH = BC // 16
HALVES = ((0, 96), (96, 104))
SH_MAX = 104
DBLK = 2
NDBLK = D_F0 // DBLK


def _sc_encoder(f0t_hbm, pht_hbm, mdt_hbm, tbl_hbm, wb_hbm, out_hbm,
                tbl_v, wb_v, f0_v, ph_v, md_v, sl0, sl1, sem0, sem1):
    wid = lax.axis_index("s") * 2 + lax.axis_index("c")
    bc0 = wid * BC

    pltpu.sync_copy(tbl_hbm, tbl_v)
    pltpu.sync_copy(wb_hbm, wb_v)

    slabs = (sl0, sl1)
    sems = (sem0, sem1)

    for sh0, sh in HALVES:
        pltpu.sync_copy(f0t_hbm.at[pl.ds(sh0, sh), pl.ds(bc0, BC)],
                        f0_v.at[pl.ds(0, sh), :])
        pltpu.sync_copy(pht_hbm.at[pl.ds(sh0, sh), pl.ds(bc0, BC)],
                        ph_v.at[pl.ds(0, sh), :])
        pltpu.sync_copy(mdt_hbm.at[pl.ds(sh0, sh), pl.ds(bc0, BC)],
                        md_v.at[pl.ds(0, sh), :])

        for sec in range(3):
            dbase = sec * D_F0

            def blk_body(i2, c, sec=sec, dbase=dbase, sh0=sh0, sh=sh):
                for j in (0, 1):
                    dblk = i2 * 2 + j
                    d0 = dbase + dblk * DBLK
                    slab = slabs[j]
                    sem = sems[j]

                    @pl.when(dblk >= 2)
                    def _():
                        pltpu.make_async_copy(
                            slab.at[:, pl.ds(0, sh), :],
                            out_hbm.at[pl.ds(d0 - 2 * DBLK, DBLK),
                                       pl.ds(sh0, sh), pl.ds(bc0, BC)],
                            sem,
                        ).wait()

                    if sec == 0:
                        wvs, bvs = [], []
                        for p in range(DBLK):
                            dv = jnp.full((16,), d0 + p, dtype=jnp.int32)
                            wvs.append(plsc.load_gather(wb_v, [dv]))
                            bvs.append(plsc.load_gather(wb_v, [dv + D_F0]))

                        def row_f0(si, c2, wvs=wvs, bvs=bvs, slab=slab):
                            fcs = [f0_v[si, pl.ds(k * 16, 16)]
                                   for k in range(NBCH)]
                            for p in range(DBLK):
                                for k in range(NBCH):
                                    slab[p, si, pl.ds(k * 16, 16)] = (
                                        fcs[k] * wvs[p] + bvs[p]
                                    )
                            return c2

                        lax.fori_loop(0, sh, row_f0, 0)
                    else:
                        lbl_v = ph_v if sec == 1 else md_v
                        stride = NUM_PHONES if sec == 1 else NUM_MIDI
                        tb = 0 if sec == 1 else MIDI_BASE
                        drel0 = d0 - dbase

                        def row_tbl(si, c2, lbl_v=lbl_v, stride=stride,
                                    tb=tb, drel0=drel0, slab=slab):
                            lcs = [lbl_v[si, pl.ds(k * 16, 16)]
                                   for k in range(NBCH)]
                            gs = []
                            for p in range(DBLK):
                                off = tb + (drel0 + p) * stride
                                for k in range(NBCH):
                                    gs.append(plsc.load_gather(
                                        tbl_v, [lcs[k] + off]))
                            for p in range(DBLK):
                                for k in range(NBCH):
                                    slab[p, si, pl.ds(k * 16, 16)] = (
                                        gs[p * NBCH + k]
                                    )
                            return c2

                        lax.fori_loop(0, sh, row_tbl, 0)

                    pltpu.async_copy(
                        slab.at[:, pl.ds(0, sh), :],
                        out_hbm.at[pl.ds(d0, DBLK), pl.ds(sh0, sh),
                                   pl.ds(bc0, BC)],
                        sem,
                    )
                return c

            lax.fori_loop(0, NDBLK // 2, blk_body, 0)

            for j in (0, 1):
                dlast = dbase + (NDBLK - 2 + j) * DBLK
                pltpu.make_async_copy(
                    slabs[j].at[:, pl.ds(0, sh), :],
                    out_hbm.at[pl.ds(dlast, DBLK), pl.ds(sh0, sh),
                               pl.ds(bc0, BC)],
                    sems[j],
                ).wait()


def kernel(f0, phone_label, phone_duration, midi_label, W_f0, b_f0,
           phone_table, midi_table):
    del phone_duration
    tbl = jnp.concatenate([
        phone_table.T.reshape(-1),
        midi_table.T.reshape(-1),
    ]).astype(jnp.float32)
    wb = jnp.concatenate([W_f0[:, 0], b_f0]).astype(jnp.float32)
    f0t = f0.astype(jnp.float32).T
    pht = phone_label.astype(jnp.int32).T
    mdt = midi_label.astype(jnp.int32).T

    mesh = plsc.VectorSubcoreMesh(core_axis_name="c", subcore_axis_name="s")
    run = functools.partial(
        pl.kernel,
        mesh=mesh,
        compiler_params=pltpu.CompilerParams(needs_layout_passes=False),
        out_type=jax.ShapeDtypeStruct((D_OUT, S, B), jnp.float32),
        scratch_types=[
            pltpu.VMEM((D_F0 * NUM_PHONES + D_F0 * NUM_MIDI,), jnp.float32),
            pltpu.VMEM((2 * D_F0,), jnp.float32),
            pltpu.VMEM((SH_MAX, BC), jnp.float32),
            pltpu.VMEM((SH_MAX, BC), jnp.int32),
            pltpu.VMEM((SH_MAX, BC), jnp.int32),
            pltpu.VMEM((DBLK, SH_MAX, BC), jnp.float32),
            pltpu.VMEM((DBLK, SH_MAX, BC), jnp.float32),
            pltpu.SemaphoreType.DMA,
            pltpu.SemaphoreType.DMA,
        ],
    )(_sc_encoder)
    out = run(f0t, pht, mdt, tbl, wb)
    return jnp.transpose(out, (2, 0, 1))
